# baseline (speedup 1.0000x reference)
import jax
import jax.numpy as jnp
from jax import lax
from jax.experimental import pallas as pl
from jax.experimental.pallas import tpu as pltpu
from jax.experimental.pallas import tpu_sc as plsc

N_TYPES = 100
NUM_RBF = 16
CUTOFF = 5.0
N_BASIS_SET = 2
NUM_EDGES = 1600000
N_PAIRS = N_TYPES * (N_TYPES + 1) // 2

_GAMMA = float((NUM_RBF / CUTOFF) ** 2)
_CENTERS = [i * (CUTOFF / (NUM_RBF - 1)) for i in range(NUM_RBF)]

_CUT_POLY = [
    1.0, -2.467400550842285, 2.0293474197387695, -0.6675792336463928,
    0.11751490086317062, -0.012679492123425007, 0.0007969553698785603,
]

_C = 256
_NCHUNKS = NUM_EDGES // _C
_L = 16


def _sc_kernel_body(d_hbm, ti_hbm, tj_hbm, w_hbm, out_hbm,
                    w_v, d_v, ti_v, tj_v, out_v):
    info = plsc.get_sparse_core_info()
    nw = info.num_cores * info.num_subcores
    wid = lax.axis_index("s") * info.num_cores + lax.axis_index("c")

    pltpu.sync_copy(w_hbm, w_v)

    lanes = lax.broadcasted_iota(jnp.int32, (_L,), 0)
    my_nchunks = (_NCHUNKS - wid + nw - 1) // nw

    def chunk_body(n, carry):
        chunk = wid + n * nw
        base = chunk * _C
        pltpu.sync_copy(d_hbm.at[pl.ds(base, _C)], d_v)
        pltpu.sync_copy(ti_hbm.at[pl.ds(base, _C)], ti_v)
        pltpu.sync_copy(tj_hbm.at[pl.ds(base, _C)], tj_v)

        def group_body(g, carry2):
            off = g * _L
            d16 = d_v[pl.ds(off, _L)]
            ti16 = ti_v[pl.ds(off, _L)]
            tj16 = tj_v[pl.ds(off, _L)]
            i_ = jnp.minimum(ti16, tj16)
            j_ = jnp.maximum(ti16, tj16)
            k16 = ((2 * N_TYPES - i_ + 1) * i_ >> 1) + (j_ - i_)
            kf = k16 * NUM_RBF
            x = d16 * (1.0 / CUTOFF)
            u = x * x
            p = jnp.full((_L,), _CUT_POLY[-1], jnp.float32)
            for c in reversed(_CUT_POLY[:-1]):
                p = p * u + c
            cut16 = jnp.where(d16 < CUTOFF, p, jnp.zeros((_L,), jnp.float32))
            rows = off + lanes
            for r in range(NUM_RBF):
                colw = plsc.load_gather(w_v, [kf + r])
                regc = jnp.minimum(jnp.maximum(colw, 0.0), 1.0)
                t = d16 - _CENTERS[r]
                gr = jnp.exp(t * t * (-_GAMMA))
                cols = jnp.full((_L,), r, jnp.int32)
                plsc.store_scatter(out_v, [rows, cols], gr * cut16 * regc)
            return carry2

        lax.fori_loop(0, _C // _L, group_body, 0)
        pltpu.sync_copy(out_v, out_hbm.at[0, pl.ds(base, _C), :])
        pltpu.sync_copy(out_v, out_hbm.at[1, pl.ds(base, _C), :])
        return carry

    lax.fori_loop(0, my_nchunks, chunk_body, 0)


def kernel(distances, type_i, type_j, w):
    mesh = plsc.VectorSubcoreMesh(core_axis_name="c", subcore_axis_name="s")
    f = pl.kernel(
        _sc_kernel_body,
        mesh=mesh,
        compiler_params=pltpu.CompilerParams(needs_layout_passes=False),
        out_type=jax.ShapeDtypeStruct((N_BASIS_SET, NUM_EDGES, NUM_RBF),
                                      jnp.float32),
        scratch_types=[
            pltpu.VMEM((N_PAIRS * NUM_RBF,), jnp.float32),
            pltpu.VMEM((_C,), jnp.float32),
            pltpu.VMEM((_C,), jnp.int32),
            pltpu.VMEM((_C,), jnp.int32),
            pltpu.VMEM((_C, NUM_RBF), jnp.float32),
        ],
    )
    return f(distances, type_i, type_j, w.reshape(-1))


# speedup vs baseline: 1.2985x; 1.2985x over previous
import jax
import jax.numpy as jnp
from jax import lax
from jax.experimental import pallas as pl
from jax.experimental.pallas import tpu as pltpu
from jax.experimental.pallas import tpu_sc as plsc

N_TYPES = 100
NUM_RBF = 16
CUTOFF = 5.0
N_BASIS_SET = 2
NUM_EDGES = 1600000
N_PAIRS = N_TYPES * (N_TYPES + 1) // 2

_GAMMA = float((NUM_RBF / CUTOFF) ** 2)
_CENTERS = [i * (CUTOFF / (NUM_RBF - 1)) for i in range(NUM_RBF)]

_CUT_POLY = [
    1.0, -2.467400550842285, 2.0293474197387695, -0.6675792336463928,
    0.11751490086317062, -0.012679492123425007, 0.0007969553698785603,
]

_C = 80
_NCHUNKS = NUM_EDGES // _C
_NW = 32
_NPW = _NCHUNKS // _NW
_L = 16


def _sc_kernel_body(d_hbm, ti_hbm, tj_hbm, w_hbm, out_hbm,
                    w_v, d0, d1, ti0, ti1, tj0, tj1, o0, o1,
                    si0, si1, so0, so1):
    info = plsc.get_sparse_core_info()
    nw = info.num_cores * info.num_subcores
    wid = lax.axis_index("s") * info.num_cores + lax.axis_index("c")

    pltpu.sync_copy(w_hbm, w_v)

    lanes = lax.broadcasted_iota(jnp.int32, (_L,), 0)
    d_b = (d0, d1)
    ti_b = (ti0, ti1)
    tj_b = (tj0, tj1)
    o_b = (o0, o1)
    si_b = (si0, si1)
    so_b = (so0, so1)

    def start_in(n, slot):
        base = (wid + n * nw) * _C
        pltpu.async_copy(d_hbm.at[pl.ds(base, _C)], d_b[slot], si_b[slot])
        pltpu.async_copy(ti_hbm.at[pl.ds(base, _C)], ti_b[slot], si_b[slot])
        pltpu.async_copy(tj_hbm.at[pl.ds(base, _C)], tj_b[slot], si_b[slot])

    def wait_in(n, slot):
        base = (wid + n * nw) * _C
        pltpu.make_async_copy(d_hbm.at[pl.ds(base, _C)], d_b[slot],
                              si_b[slot]).wait()
        pltpu.make_async_copy(ti_hbm.at[pl.ds(base, _C)], ti_b[slot],
                              si_b[slot]).wait()
        pltpu.make_async_copy(tj_hbm.at[pl.ds(base, _C)], tj_b[slot],
                              si_b[slot]).wait()

    def start_out(n, slot):
        base = (wid + n * nw) * _C
        pltpu.async_copy(o_b[slot], out_hbm.at[0, pl.ds(base, _C), :],
                         so_b[slot])
        pltpu.async_copy(o_b[slot], out_hbm.at[1, pl.ds(base, _C), :],
                         so_b[slot])

    def wait_out(n, slot):
        base = (wid + n * nw) * _C
        pltpu.make_async_copy(o_b[slot], out_hbm.at[0, pl.ds(base, _C), :],
                              so_b[slot]).wait()
        pltpu.make_async_copy(o_b[slot], out_hbm.at[1, pl.ds(base, _C), :],
                              so_b[slot]).wait()

    def compute(slot):
        d_v, ti_v, tj_v, out_v = d_b[slot], ti_b[slot], tj_b[slot], o_b[slot]
        for g in range(_C // _L):
            off = g * _L
            d16 = d_v[pl.ds(off, _L)]
            ti16 = ti_v[pl.ds(off, _L)]
            tj16 = tj_v[pl.ds(off, _L)]
            i_ = jnp.minimum(ti16, tj16)
            j_ = jnp.maximum(ti16, tj16)
            k16 = ((2 * N_TYPES - i_ + 1) * i_ >> 1) + (j_ - i_)
            kf = k16 * NUM_RBF
            x = d16 * (1.0 / CUTOFF)
            u = x * x
            p = jnp.full((_L,), _CUT_POLY[-1], jnp.float32)
            for c in reversed(_CUT_POLY[:-1]):
                p = p * u + c
            cut16 = jnp.where(d16 < CUTOFF, p, jnp.zeros((_L,), jnp.float32))
            rows = off + lanes
            for r in range(NUM_RBF):
                colw = plsc.load_gather(w_v, [kf + r])
                regc = jnp.minimum(jnp.maximum(colw, 0.0), 1.0)
                t = d16 - _CENTERS[r]
                gr = jnp.exp(t * t * (-_GAMMA))
                cols = jnp.full((_L,), r, jnp.int32)
                plsc.store_scatter(out_v, [rows, cols], gr * cut16 * regc)

    start_in(0, 0)

    def pair_body(pidx, carry):
        for slot in (0, 1):
            n = 2 * pidx + slot
            start_in(n + 1, 1 - slot)
            wait_in(n, slot)

            @pl.when(pidx > 0)
            def _():
                wait_out(n - 2, slot)

            compute(slot)
            start_out(n, slot)
        return carry

    lax.fori_loop(0, _NPW // 2, pair_body, 0)

    n_last = _NPW - 1
    wait_in(n_last, 0)
    wait_out(n_last - 2, 0)
    compute(0)
    start_out(n_last, 0)
    wait_out(n_last - 1, 1)
    wait_out(n_last, 0)


def kernel(distances, type_i, type_j, w):
    mesh = plsc.VectorSubcoreMesh(core_axis_name="c", subcore_axis_name="s")
    f = pl.kernel(
        _sc_kernel_body,
        mesh=mesh,
        compiler_params=pltpu.CompilerParams(needs_layout_passes=False),
        out_type=jax.ShapeDtypeStruct((N_BASIS_SET, NUM_EDGES, NUM_RBF),
                                      jnp.float32),
        scratch_types=[
            pltpu.VMEM((N_PAIRS * NUM_RBF,), jnp.float32),
            pltpu.VMEM((_C,), jnp.float32),
            pltpu.VMEM((_C,), jnp.float32),
            pltpu.VMEM((_C,), jnp.int32),
            pltpu.VMEM((_C,), jnp.int32),
            pltpu.VMEM((_C,), jnp.int32),
            pltpu.VMEM((_C,), jnp.int32),
            pltpu.VMEM((_C, NUM_RBF), jnp.float32),
            pltpu.VMEM((_C, NUM_RBF), jnp.float32),
            pltpu.SemaphoreType.DMA,
            pltpu.SemaphoreType.DMA,
            pltpu.SemaphoreType.DMA,
            pltpu.SemaphoreType.DMA,
        ],
    )
    return f(distances, type_i, type_j, w.reshape(-1))


# speedup vs baseline: 1.6531x; 1.2730x over previous
import jax
import jax.numpy as jnp
from jax import lax
from jax.experimental import pallas as pl
from jax.experimental.pallas import tpu as pltpu
from jax.experimental.pallas import tpu_sc as plsc

N_TYPES = 100
NUM_RBF = 16
CUTOFF = 5.0
N_BASIS_SET = 2
NUM_EDGES = 1600000
N_PAIRS = N_TYPES * (N_TYPES + 1) // 2
_WROW = NUM_RBF + 1

_GAMMA = float((NUM_RBF / CUTOFF) ** 2)

_CUT_POLY = [
    1.0, -2.467400550842285, 2.0293474197387695, -0.6675792336463928,
    0.11751490086317062, -0.012679492123425007, 0.0007969553698785603,
]

_TAKE_DNUMS = lax.GatherDimensionNumbers(
    offset_dims=(), collapsed_slice_dims=(0,), start_index_map=(0,))


def _lane_take(x, ce):
    return lax.gather(x, ce[:, None], _TAKE_DNUMS, (1,),
                      mode=lax.GatherScatterMode.PROMISE_IN_BOUNDS)


_C = 80
_NCHUNKS = NUM_EDGES // _C
_NW = 32
_NPW = _NCHUNKS // _NW
_L = 16


def _sc_kernel_body(d_hbm, ti_hbm, tj_hbm, w_hbm, out_hbm,
                    w_v, d0, d1, ti0, ti1, tj0, tj1, o0, o1,
                    si0, si1, so0, so1):
    info = plsc.get_sparse_core_info()
    nw = info.num_cores * info.num_subcores
    wid = lax.axis_index("s") * info.num_cores + lax.axis_index("c")

    pltpu.sync_copy(w_hbm, w_v)

    lanes = lax.broadcasted_iota(jnp.int32, (_L,), 0)
    centers = lanes.astype(jnp.float32) * (CUTOFF / (NUM_RBF - 1))
    d_b = (d0, d1)
    ti_b = (ti0, ti1)
    tj_b = (tj0, tj1)
    o_b = (o0, o1)
    si_b = (si0, si1)
    so_b = (so0, so1)

    def start_in(n, slot):
        base = (wid + n * nw) * _C
        pltpu.async_copy(d_hbm.at[pl.ds(base, _C)], d_b[slot], si_b[slot])
        pltpu.async_copy(ti_hbm.at[pl.ds(base, _C)], ti_b[slot], si_b[slot])
        pltpu.async_copy(tj_hbm.at[pl.ds(base, _C)], tj_b[slot], si_b[slot])

    def wait_in(n, slot):
        base = (wid + n * nw) * _C
        pltpu.make_async_copy(d_hbm.at[pl.ds(base, _C)], d_b[slot],
                              si_b[slot]).wait()
        pltpu.make_async_copy(ti_hbm.at[pl.ds(base, _C)], ti_b[slot],
                              si_b[slot]).wait()
        pltpu.make_async_copy(tj_hbm.at[pl.ds(base, _C)], tj_b[slot],
                              si_b[slot]).wait()

    def start_out(n, slot):
        base = (wid + n * nw) * _C
        pltpu.async_copy(o_b[slot], out_hbm.at[0, pl.ds(base, _C), :],
                         so_b[slot])
        pltpu.async_copy(o_b[slot], out_hbm.at[1, pl.ds(base, _C), :],
                         so_b[slot])

    def wait_out(n, slot):
        base = (wid + n * nw) * _C
        pltpu.make_async_copy(o_b[slot], out_hbm.at[0, pl.ds(base, _C), :],
                              so_b[slot]).wait()
        pltpu.make_async_copy(o_b[slot], out_hbm.at[1, pl.ds(base, _C), :],
                              so_b[slot]).wait()

    def compute(slot):
        d_v, ti_v, tj_v, out_v = d_b[slot], ti_b[slot], tj_b[slot], o_b[slot]
        for g in range(_C // _L):
            off = g * _L
            d16 = d_v[pl.ds(off, _L)]
            ti16 = ti_v[pl.ds(off, _L)]
            tj16 = tj_v[pl.ds(off, _L)]
            i_ = jnp.minimum(ti16, tj16)
            j_ = jnp.maximum(ti16, tj16)
            k16 = ((2 * N_TYPES - i_ + 1) * i_ >> 1) + (j_ - i_)
            k17 = k16 * _WROW
            x = d16 * (1.0 / CUTOFF)
            u = x * x
            p = jnp.full((_L,), _CUT_POLY[-1], jnp.float32)
            for c in reversed(_CUT_POLY[:-1]):
                p = p * u + c
            cut16 = jnp.where(d16 < CUTOFF, p, jnp.zeros((_L,), jnp.float32))
            for e in range(_L):
                ce = jnp.full((_L,), e, jnp.int32)
                dB = _lane_take(d16, ce)
                cB = _lane_take(cut16, ce)
                kB = _lane_take(k17, ce)
                row = plsc.load_gather(w_v, [kB + lanes])
                regc = jnp.minimum(jnp.maximum(row, 0.0), 1.0)
                t = dB - centers
                gauss = jnp.exp(t * t * (-_GAMMA))
                out_v[off + e, :] = gauss * cB * regc
        return

    start_in(0, 0)

    def pair_body(pidx, carry):
        for slot in (0, 1):
            n = 2 * pidx + slot
            start_in(n + 1, 1 - slot)
            wait_in(n, slot)

            @pl.when(pidx > 0)
            def _():
                wait_out(n - 2, slot)

            compute(slot)
            start_out(n, slot)
        return carry

    lax.fori_loop(0, _NPW // 2, pair_body, 0)

    n_last = _NPW - 1
    wait_in(n_last, 0)
    wait_out(n_last - 2, 0)
    compute(0)
    start_out(n_last, 0)
    wait_out(n_last - 1, 1)
    wait_out(n_last, 0)


def kernel(distances, type_i, type_j, w):
    mesh = plsc.VectorSubcoreMesh(core_axis_name="c", subcore_axis_name="s")
    f = pl.kernel(
        _sc_kernel_body,
        mesh=mesh,
        compiler_params=pltpu.CompilerParams(needs_layout_passes=False),
        out_type=jax.ShapeDtypeStruct((N_BASIS_SET, NUM_EDGES, NUM_RBF),
                                      jnp.float32),
        scratch_types=[
            pltpu.VMEM((N_PAIRS * _WROW,), jnp.float32),
            pltpu.VMEM((_C,), jnp.float32),
            pltpu.VMEM((_C,), jnp.float32),
            pltpu.VMEM((_C,), jnp.int32),
            pltpu.VMEM((_C,), jnp.int32),
            pltpu.VMEM((_C,), jnp.int32),
            pltpu.VMEM((_C,), jnp.int32),
            pltpu.VMEM((_C, NUM_RBF), jnp.float32),
            pltpu.VMEM((_C, NUM_RBF), jnp.float32),
            pltpu.SemaphoreType.DMA,
            pltpu.SemaphoreType.DMA,
            pltpu.SemaphoreType.DMA,
            pltpu.SemaphoreType.DMA,
        ],
    )
    w_pad = jnp.pad(w, ((0, 0), (0, _WROW - NUM_RBF)))
    return f(distances, type_i, type_j, w_pad.reshape(-1))
